# baseline (speedup 1.0000x reference)
import functools

import jax
import jax.numpy as jnp
from jax import lax
from jax.experimental import pallas as pl
from jax.experimental.pallas import tpu as pltpu
from jax.experimental.pallas import tpu_sc as plsc

CAT = 1000
D = 200
BATCH = 16384
AW = 640
FCE_OFF = 604
NC, NS = 2, 16
NW = NC * NS
ROWS_PER_W = BATCH // NW
CHUNK = 32
NCHUNK = ROWS_PER_W // CHUNK


def _prep_body(wsim, gsim, mix, wrep, wexp, fce, out_ref):
    a_sim = mix[:, 0:1] * wsim[...] + mix[:, 1:2] * gsim[...]
    z4 = jnp.zeros((CAT, 4), jnp.float32)
    pad = jnp.zeros((CAT, AW - FCE_OFF - 12), jnp.float32)
    out_ref[...] = jnp.concatenate(
        [a_sim, wrep[...], wexp[...], z4, fce[0:CAT, :], pad], axis=1)


def _build_fused_table(wsim, gsim, mix, wrep, wexp, fce):
    return pl.pallas_call(
        _prep_body,
        out_shape=jax.ShapeDtypeStruct((CAT, AW), jnp.float32),
    )(wsim, gsim, mix, wrep, wexp, fce)


def _sc_body(cat_hbm, seq_hbm, table_hbm, grep_hbm, gexp_hbm,
             fs_hbm, jsim_hbm,
             idx_v, a_v0, a_v1, sim_v0, sim_v1, rep_v0, rep_v1, exp_v0, exp_v1,
             grep_v, gexp_v, fs_v0, fs_v1, res_v0, res_v1,
             gsem0, gsem1, ssem0, ssem1, osem0, osem1):
    wid = lax.axis_index("s") * NC + lax.axis_index("c")
    a_vs = (a_v0, a_v1)
    sim_vs = (sim_v0, sim_v1)
    rep_vs = (rep_v0, rep_v1)
    exp_vs = (exp_v0, exp_v1)
    fs_vs = (fs_v0, fs_v1)
    res_vs = (res_v0, res_v1)
    gsems = (gsem0, gsem1)
    ssems = (ssem0, ssem1)
    osems = (osem0, osem1)
    tail = jnp.where(lax.iota(jnp.int32, 16) >= 8,
                     jnp.float32(1.0), jnp.float32(0.0))
    lanes = lax.iota(jnp.int32, 16)
    pltpu.sync_copy(grep_hbm, grep_v)
    pltpu.sync_copy(gexp_hbm, gexp_v)
    pltpu.sync_copy(cat_hbm.at[pl.ds(wid * ROWS_PER_W, ROWS_PER_W)], idx_v)
    gr = [grep_v[pl.ds(16 * j, 16)] for j in range(12)]
    ge = [gexp_v[pl.ds(16 * j, 16)] for j in range(12)]
    gr.append(grep_v[pl.ds(184, 16)] * tail)
    ge.append(gexp_v[pl.ds(184, 16)] * tail)

    def in_copies(buf, ci):
        base = wid * ROWS_PER_W + ci * CHUNK
        return (
            pltpu.make_async_copy(
                table_hbm.at[idx_v.at[pl.ds(ci * CHUNK, CHUNK)]],
                a_vs[buf], gsems[buf]),
            pltpu.make_async_copy(seq_hbm.at[pl.ds(base * D, CHUNK * D)],
                                  sim_vs[buf], ssems[buf]),
            pltpu.make_async_copy(
                seq_hbm.at[pl.ds(BATCH * D + base * D, CHUNK * D)],
                rep_vs[buf], ssems[buf]),
            pltpu.make_async_copy(
                seq_hbm.at[pl.ds(2 * BATCH * D + base * D, CHUNK * D)],
                exp_vs[buf], ssems[buf]),
        )

    def start_in(buf, ci):
        for c in in_copies(buf, ci):
            c.start()

    def wait_in(buf, ci):
        for c in in_copies(buf, ci):
            c.wait()

    def drain_out(buf):
        base = wid * ROWS_PER_W
        pltpu.make_async_copy(fs_hbm.at[pl.ds(base * 16, CHUNK * 16)],
                              fs_vs[buf], osems[buf]).wait()
        pltpu.make_async_copy(jsim_hbm.at[pl.ds(base, CHUNK)], res_vs[buf],
                              osems[buf]).wait()

    def compute_chunk(buf, ci):
        base = wid * ROWS_PER_W + ci * CHUNK
        a_v, sim_v, rep_v, exp_v = a_vs[buf], sim_vs[buf], rep_vs[buf], exp_vs[buf]
        fs_v, res_v = fs_vs[buf], res_vs[buf]

        def grp_body(g, gcarry):
            def row_body(k, dsim):
                r = g * 16 + k
                acc_s = jnp.zeros((16,), jnp.float32)
                acc_r1 = jnp.zeros((16,), jnp.float32)
                acc_r2 = jnp.zeros((16,), jnp.float32)
                acc_e1 = jnp.zeros((16,), jnp.float32)
                acc_e2 = jnp.zeros((16,), jnp.float32)
                rb = r * D
                for j in range(12):
                    o = 16 * j
                    sv = sim_v[pl.ds(rb + o, 16)]
                    rv = rep_v[pl.ds(rb + o, 16)]
                    ev = exp_v[pl.ds(rb + o, 16)]
                    acc_s = acc_s + sv * a_v[r, pl.ds(o, 16)]
                    acc_r1 = acc_r1 + rv * a_v[r, pl.ds(D + o, 16)]
                    acc_r2 = acc_r2 + rv * gr[j]
                    acc_e1 = acc_e1 + ev * a_v[r, pl.ds(2 * D + o, 16)]
                    acc_e2 = acc_e2 + ev * ge[j]
                o = 184
                sv = sim_v[pl.ds(rb + o, 16)] * tail
                rv = rep_v[pl.ds(rb + o, 16)]
                ev = exp_v[pl.ds(rb + o, 16)]
                acc_s = acc_s + sv * a_v[r, pl.ds(o, 16)]
                acc_r1 = acc_r1 + rv * tail * a_v[r, pl.ds(D + o, 16)]
                acc_r2 = acc_r2 + rv * gr[12]
                acc_e1 = acc_e1 + ev * tail * a_v[r, pl.ds(2 * D + o, 16)]
                acc_e2 = acc_e2 + ev * ge[12]
                fsrow = a_v[r, pl.ds(FCE_OFF - 4, 16)]
                fsrow = jnp.where(jnp.equal(lanes, 0), jnp.sum(acc_r1), fsrow)
                fsrow = jnp.where(jnp.equal(lanes, 1), jnp.sum(acc_r2), fsrow)
                fsrow = jnp.where(jnp.equal(lanes, 2), jnp.sum(acc_e1), fsrow)
                fsrow = jnp.where(jnp.equal(lanes, 3), jnp.sum(acc_e2), fsrow)
                fs_v[pl.ds(r * 16, 16)] = fsrow
                dsim = jnp.where(jnp.equal(lanes, k), jnp.sum(acc_s), dsim)
                return dsim

            dsim = lax.fori_loop(0, 16, row_body, jnp.zeros((16,), jnp.float32))
            res_v[pl.ds(16 * g, 16)] = dsim
            return gcarry

        lax.fori_loop(0, CHUNK // 16, grp_body, 0)
        pltpu.async_copy(fs_v, fs_hbm.at[pl.ds(base * 16, CHUNK * 16)],
                         osems[buf])
        pltpu.async_copy(res_v, jsim_hbm.at[pl.ds(base, CHUNK)], osems[buf])

    start_in(0, 0)
    start_in(1, 1)

    def pair_body(t, carry):
        for buf in (0, 1):
            ci = 2 * t + buf
            wait_in(buf, ci)

            @pl.when(t >= 1)
            def _():
                drain_out(buf)

            compute_chunk(buf, ci)

            @pl.when(ci + 2 < NCHUNK)
            def _():
                start_in(buf, ci + 2)

        return carry

    lax.fori_loop(0, NCHUNK // 2, pair_body, 0)
    drain_out(0)
    drain_out(1)


_sc_call = functools.partial(
    pl.kernel,
    out_type=(jax.ShapeDtypeStruct((BATCH * 16,), jnp.float32),
              jax.ShapeDtypeStruct((BATCH,), jnp.float32)),
    mesh=plsc.VectorSubcoreMesh(core_axis_name="c", subcore_axis_name="s",
                                num_cores=NC, num_subcores=NS),
    scratch_types=[
        pltpu.VMEM((ROWS_PER_W,), jnp.int32),
        pltpu.VMEM((CHUNK, AW), jnp.float32),
        pltpu.VMEM((CHUNK, AW), jnp.float32),
        pltpu.VMEM((CHUNK * D,), jnp.float32),
        pltpu.VMEM((CHUNK * D,), jnp.float32),
        pltpu.VMEM((CHUNK * D,), jnp.float32),
        pltpu.VMEM((CHUNK * D,), jnp.float32),
        pltpu.VMEM((CHUNK * D,), jnp.float32),
        pltpu.VMEM((CHUNK * D,), jnp.float32),
        pltpu.VMEM((D,), jnp.float32),
        pltpu.VMEM((D,), jnp.float32),
        pltpu.VMEM((CHUNK * 16,), jnp.float32),
        pltpu.VMEM((CHUNK * 16,), jnp.float32),
        pltpu.VMEM((CHUNK,), jnp.float32),
        pltpu.VMEM((CHUNK,), jnp.float32),
        pltpu.SemaphoreType.DMA,
        pltpu.SemaphoreType.DMA,
        pltpu.SemaphoreType.DMA,
        pltpu.SemaphoreType.DMA,
        pltpu.SemaphoreType.DMA,
        pltpu.SemaphoreType.DMA,
    ],
    compiler_params=pltpu.CompilerParams(needs_layout_passes=False,
                                         use_tc_tiling_on_sc=False),
)(_sc_body)


def _mlp_body(fs, jsim, w1, b1, w2, b2, out_ref):
    h = jnp.dot(fs[...], w1[...], preferred_element_type=jnp.float32) + b1[...]
    jf = jnp.dot(h, w2[...], preferred_element_type=jnp.float32) + b2[...]
    out_ref[...] = jax.nn.sigmoid(jf) * jsim[...]


def _mlp_call(fs, jsim, w1, b1, w2, b2):
    return pl.pallas_call(
        _mlp_body,
        out_shape=jax.ShapeDtypeStruct((BATCH, 1), jnp.float32),
    )(fs, jsim, w1, b1, w2, b2)


def kernel(item, cat, sim_seq, rep_seq, expl_seq, cat_emb, pos_cat_sim_weight,
           pos_global_sim_weight, sim_mix_emb, incat_rep_pos_weight,
           global_rep_pos_weight, incat_exp_pos_weight, global_exp_pos_weight,
           freq_cat_emb, W1, b1, W2, b2):
    table = _build_fused_table(
        pos_cat_sim_weight, pos_global_sim_weight, sim_mix_emb,
        incat_rep_pos_weight, incat_exp_pos_weight, freq_cat_emb)
    seqs = jnp.concatenate([sim_seq.reshape(-1), rep_seq.reshape(-1),
                            expl_seq.reshape(-1)])
    fs, jsim = _sc_call(cat.astype(jnp.int32), seqs, table,
                        global_rep_pos_weight.reshape(-1),
                        global_exp_pos_weight.reshape(-1))
    return _mlp_call(fs.reshape(BATCH, 16), jsim[:, None], W1, b1[None, :],
                     W2, b2[None, :])


# speedup vs baseline: 1.1616x; 1.1616x over previous
import functools

import jax
import jax.numpy as jnp
from jax import lax
from jax.experimental import pallas as pl
from jax.experimental.pallas import tpu as pltpu
from jax.experimental.pallas import tpu_sc as plsc

CAT = 1000
D = 200
BATCH = 16384
AW = 640
FCE_OFF = 604
NC, NS = 2, 16
NW = NC * NS
ROWS_PER_W = BATCH // NW
CHUNK = 32
NCHUNK = ROWS_PER_W // CHUNK


def _prep_body(wsim, gsim, mix, wrep, wexp, fce, out_ref):
    a_sim = mix[:, 0:1] * wsim[...] + mix[:, 1:2] * gsim[...]
    z4 = jnp.zeros((CAT, 4), jnp.float32)
    pad = jnp.zeros((CAT, AW - FCE_OFF - 12), jnp.float32)
    out_ref[...] = jnp.concatenate(
        [a_sim, wrep[...], wexp[...], z4, fce[0:CAT, :], pad], axis=1)


def _build_fused_table(wsim, gsim, mix, wrep, wexp, fce):
    return pl.pallas_call(
        _prep_body,
        out_shape=jax.ShapeDtypeStruct((CAT, AW), jnp.float32),
    )(wsim, gsim, mix, wrep, wexp, fce)


def _sc_body(cat_hbm, sim_hbm, rep_hbm, exp_hbm, table_hbm, grep_hbm, gexp_hbm,
             fs_hbm, jsim_hbm,
             idx_v, a_v0, a_v1, sim_v0, sim_v1, rep_v0, rep_v1, exp_v0, exp_v1,
             grep_v, gexp_v, fs_v0, fs_v1, res_v0, res_v1,
             gsem0, gsem1, ssem0, ssem1, osem0, osem1):
    wid = lax.axis_index("s") * NC + lax.axis_index("c")
    a_vs = (a_v0, a_v1)
    sim_vs = (sim_v0, sim_v1)
    rep_vs = (rep_v0, rep_v1)
    exp_vs = (exp_v0, exp_v1)
    fs_vs = (fs_v0, fs_v1)
    res_vs = (res_v0, res_v1)
    gsems = (gsem0, gsem1)
    ssems = (ssem0, ssem1)
    osems = (osem0, osem1)
    tail = jnp.where(lax.iota(jnp.int32, 16) >= 8,
                     jnp.float32(1.0), jnp.float32(0.0))
    lanes = lax.iota(jnp.int32, 16)
    pltpu.sync_copy(grep_hbm, grep_v)
    pltpu.sync_copy(gexp_hbm, gexp_v)
    pltpu.sync_copy(cat_hbm.at[pl.ds(wid * ROWS_PER_W, ROWS_PER_W)], idx_v)
    gr = [grep_v[pl.ds(16 * j, 16)] for j in range(12)]
    ge = [gexp_v[pl.ds(16 * j, 16)] for j in range(12)]
    gr.append(grep_v[pl.ds(184, 16)] * tail)
    ge.append(gexp_v[pl.ds(184, 16)] * tail)

    def in_copies(buf, ci):
        base = wid * ROWS_PER_W + ci * CHUNK
        return (
            pltpu.make_async_copy(
                table_hbm.at[idx_v.at[pl.ds(ci * CHUNK, CHUNK)]],
                a_vs[buf], gsems[buf]),
            pltpu.make_async_copy(sim_hbm.at[pl.ds(base * D, CHUNK * D)],
                                  sim_vs[buf], ssems[buf]),
            pltpu.make_async_copy(rep_hbm.at[pl.ds(base * D, CHUNK * D)],
                                  rep_vs[buf], ssems[buf]),
            pltpu.make_async_copy(exp_hbm.at[pl.ds(base * D, CHUNK * D)],
                                  exp_vs[buf], ssems[buf]),
        )

    def start_in(buf, ci):
        for c in in_copies(buf, ci):
            c.start()

    def wait_in(buf, ci):
        for c in in_copies(buf, ci):
            c.wait()

    def drain_out(buf):
        base = wid * ROWS_PER_W
        pltpu.make_async_copy(fs_hbm.at[pl.ds(base * 16, CHUNK * 16)],
                              fs_vs[buf], osems[buf]).wait()
        pltpu.make_async_copy(jsim_hbm.at[pl.ds(base, CHUNK)], res_vs[buf],
                              osems[buf]).wait()

    def compute_chunk(buf, ci):
        base = wid * ROWS_PER_W + ci * CHUNK
        a_v, sim_v, rep_v, exp_v = a_vs[buf], sim_vs[buf], rep_vs[buf], exp_vs[buf]
        fs_v, res_v = fs_vs[buf], res_vs[buf]

        def grp_body(g, gcarry):
            def row_body(k, dsim):
                r = g * 16 + k
                acc_s = jnp.zeros((16,), jnp.float32)
                acc_r1 = jnp.zeros((16,), jnp.float32)
                acc_r2 = jnp.zeros((16,), jnp.float32)
                acc_e1 = jnp.zeros((16,), jnp.float32)
                acc_e2 = jnp.zeros((16,), jnp.float32)
                rb = r * D
                for j in range(12):
                    o = 16 * j
                    sv = sim_v[pl.ds(rb + o, 16)]
                    rv = rep_v[pl.ds(rb + o, 16)]
                    ev = exp_v[pl.ds(rb + o, 16)]
                    acc_s = acc_s + sv * a_v[r, pl.ds(o, 16)]
                    acc_r1 = acc_r1 + rv * a_v[r, pl.ds(D + o, 16)]
                    acc_r2 = acc_r2 + rv * gr[j]
                    acc_e1 = acc_e1 + ev * a_v[r, pl.ds(2 * D + o, 16)]
                    acc_e2 = acc_e2 + ev * ge[j]
                o = 184
                sv = sim_v[pl.ds(rb + o, 16)] * tail
                rv = rep_v[pl.ds(rb + o, 16)]
                ev = exp_v[pl.ds(rb + o, 16)]
                acc_s = acc_s + sv * a_v[r, pl.ds(o, 16)]
                acc_r1 = acc_r1 + rv * tail * a_v[r, pl.ds(D + o, 16)]
                acc_r2 = acc_r2 + rv * gr[12]
                acc_e1 = acc_e1 + ev * tail * a_v[r, pl.ds(2 * D + o, 16)]
                acc_e2 = acc_e2 + ev * ge[12]
                fsrow = a_v[r, pl.ds(FCE_OFF - 4, 16)]
                fsrow = jnp.where(jnp.equal(lanes, 0), jnp.sum(acc_r1), fsrow)
                fsrow = jnp.where(jnp.equal(lanes, 1), jnp.sum(acc_r2), fsrow)
                fsrow = jnp.where(jnp.equal(lanes, 2), jnp.sum(acc_e1), fsrow)
                fsrow = jnp.where(jnp.equal(lanes, 3), jnp.sum(acc_e2), fsrow)
                fs_v[pl.ds(r * 16, 16)] = fsrow
                dsim = jnp.where(jnp.equal(lanes, k), jnp.sum(acc_s), dsim)
                return dsim

            dsim = lax.fori_loop(0, 16, row_body, jnp.zeros((16,), jnp.float32))
            res_v[pl.ds(16 * g, 16)] = dsim
            return gcarry

        lax.fori_loop(0, CHUNK // 16, grp_body, 0)
        pltpu.async_copy(fs_v, fs_hbm.at[pl.ds(base * 16, CHUNK * 16)],
                         osems[buf])
        pltpu.async_copy(res_v, jsim_hbm.at[pl.ds(base, CHUNK)], osems[buf])

    start_in(0, 0)
    start_in(1, 1)

    def pair_body(t, carry):
        for buf in (0, 1):
            ci = 2 * t + buf
            wait_in(buf, ci)

            @pl.when(t >= 1)
            def _():
                drain_out(buf)

            compute_chunk(buf, ci)

            @pl.when(ci + 2 < NCHUNK)
            def _():
                start_in(buf, ci + 2)

        return carry

    lax.fori_loop(0, NCHUNK // 2, pair_body, 0)
    drain_out(0)
    drain_out(1)


_sc_call = functools.partial(
    pl.kernel,
    out_type=(jax.ShapeDtypeStruct((BATCH * 16,), jnp.float32),
              jax.ShapeDtypeStruct((BATCH,), jnp.float32)),
    mesh=plsc.VectorSubcoreMesh(core_axis_name="c", subcore_axis_name="s",
                                num_cores=NC, num_subcores=NS),
    scratch_types=[
        pltpu.VMEM((ROWS_PER_W,), jnp.int32),
        pltpu.VMEM((CHUNK, AW), jnp.float32),
        pltpu.VMEM((CHUNK, AW), jnp.float32),
        pltpu.VMEM((CHUNK * D,), jnp.float32),
        pltpu.VMEM((CHUNK * D,), jnp.float32),
        pltpu.VMEM((CHUNK * D,), jnp.float32),
        pltpu.VMEM((CHUNK * D,), jnp.float32),
        pltpu.VMEM((CHUNK * D,), jnp.float32),
        pltpu.VMEM((CHUNK * D,), jnp.float32),
        pltpu.VMEM((D,), jnp.float32),
        pltpu.VMEM((D,), jnp.float32),
        pltpu.VMEM((CHUNK * 16,), jnp.float32),
        pltpu.VMEM((CHUNK * 16,), jnp.float32),
        pltpu.VMEM((CHUNK,), jnp.float32),
        pltpu.VMEM((CHUNK,), jnp.float32),
        pltpu.SemaphoreType.DMA,
        pltpu.SemaphoreType.DMA,
        pltpu.SemaphoreType.DMA,
        pltpu.SemaphoreType.DMA,
        pltpu.SemaphoreType.DMA,
        pltpu.SemaphoreType.DMA,
    ],
    compiler_params=pltpu.CompilerParams(needs_layout_passes=False,
                                         use_tc_tiling_on_sc=False),
)(_sc_body)


def _mlp_body(fs, jsim, w1, b1, w2, b2, out_ref):
    h = jnp.dot(fs[...], w1[...], preferred_element_type=jnp.float32) + b1[...]
    jf = jnp.dot(h, w2[...], preferred_element_type=jnp.float32) + b2[...]
    out_ref[...] = jax.nn.sigmoid(jf) * jsim[...]


def _mlp_call(fs, jsim, w1, b1, w2, b2):
    return pl.pallas_call(
        _mlp_body,
        out_shape=jax.ShapeDtypeStruct((BATCH, 1), jnp.float32),
    )(fs, jsim, w1, b1, w2, b2)


def kernel(item, cat, sim_seq, rep_seq, expl_seq, cat_emb, pos_cat_sim_weight,
           pos_global_sim_weight, sim_mix_emb, incat_rep_pos_weight,
           global_rep_pos_weight, incat_exp_pos_weight, global_exp_pos_weight,
           freq_cat_emb, W1, b1, W2, b2):
    table = _build_fused_table(
        pos_cat_sim_weight, pos_global_sim_weight, sim_mix_emb,
        incat_rep_pos_weight, incat_exp_pos_weight, freq_cat_emb)
    fs, jsim = _sc_call(cat.astype(jnp.int32), sim_seq.reshape(-1),
                        rep_seq.reshape(-1), expl_seq.reshape(-1), table,
                        global_rep_pos_weight.reshape(-1),
                        global_exp_pos_weight.reshape(-1))
    return _mlp_call(fs.reshape(BATCH, 16), jsim[:, None], W1, b1[None, :],
                     W2, b2[None, :])
